# baseline (speedup 1.0000x reference)
import functools

import jax
import jax.numpy as jnp
from jax import lax
from jax.experimental import pallas as pl
from jax.experimental.pallas import tpu as pltpu
from jax.experimental.pallas import tpu_sc as plsc

_VOCAB = 1000000
_D = 32
_B = 16384

_info = plsc.get_sparse_core_info()
_NC, _NS = _info.num_cores, _info.num_subcores
_NW = _NC * _NS
_BPW = _B // _NW
_RING = 8

_mesh = plsc.VectorSubcoreMesh(core_axis_name="c", subcore_axis_name="s")


@functools.partial(
    pl.kernel,
    mesh=_mesh,
    out_type=jax.ShapeDtypeStruct((_D, _B), jnp.float32),
    scratch_types=[
        pltpu.VMEM((_BPW,), jnp.int32),
        pltpu.VMEM((_RING, 32, 128), jnp.float32),
        pltpu.VMEM((4, 8, _BPW), jnp.float32),
        pltpu.SemaphoreType.DMA((_RING,)),
    ],
    compiler_params=pltpu.CompilerParams(
        use_tc_tiling_on_sc=True, needs_layout_passes=False
    ),
)
def _sc_gather(table_hbm, idx_hbm, out_hbm, idx_v, blk_v, cols_v, sems):
    wid = lax.axis_index("s") * _NC + lax.axis_index("c")
    base = wid * _BPW
    pltpu.sync_copy(idx_hbm.at[pl.ds(base, _BPW)], idx_v)

    lane = lax.broadcasted_iota(jnp.int32, (16,), 0)

    def read_idx(i):
        chunk = idx_v[pl.ds((i >> 4) * 16, 16)]
        return jnp.sum(jnp.where(lane == (i & 15), chunk, 0))

    def fetch(slot, v):
        c0 = (v >> 7) * 128
        pltpu.async_copy(
            table_hbm.at[:, pl.ds(c0, 128)],
            blk_v.at[slot],
            sems.at[slot],
        )

    def wait_block(slot):
        pltpu.make_async_copy(
            table_hbm.at[:, pl.ds(0, 128)],
            blk_v.at[slot],
            sems.at[slot],
        ).wait()

    for i in range(_RING):
        fetch(i, read_idx(jnp.int32(i)))

    def body(i, carry):
        slot = i & (_RING - 1)
        wait_block(slot)
        v = read_idx(i)
        l16 = jnp.full((16,), v & 127, dtype=jnp.int32)
        i16 = jnp.full((16,), i, dtype=jnp.int32)
        zero_dep = jnp.int32(0)
        for h in range(2):
            d16 = lane + (16 * h)
            vals = plsc.load_gather(blk_v.at[slot], [d16, l16])
            plsc.store_scatter(cols_v, [d16 >> 3, d16 & 7, i16], vals)
            zero_dep = zero_dep | (jnp.sum(plsc.bitcast(vals, jnp.int32)) & 0)
        nxt = i + _RING

        @pl.when(nxt < _BPW)
        def _():
            fetch(slot, read_idx(nxt) | zero_dep)

        return carry

    lax.fori_loop(0, _BPW, body, 0)

    for r in range(4):
        pltpu.sync_copy(
            cols_v.at[r], out_hbm.at[pl.ds(r * 8, 8), pl.ds(base, _BPW)]
        )


def kernel(x, embeddings):
    out_t = _sc_gather(embeddings.T, x.astype(jnp.int32))
    return out_t.T


# speedup vs baseline: 1.0008x; 1.0008x over previous
import functools

import jax
import jax.numpy as jnp
from jax import lax
from jax.experimental import pallas as pl
from jax.experimental.pallas import tpu as pltpu
from jax.experimental.pallas import tpu_sc as plsc

_VOCAB = 1000000
_D = 32
_B = 16384

_info = plsc.get_sparse_core_info()
_NC, _NS = _info.num_cores, _info.num_subcores
_NW = _NC * _NS
_BPW = _B // _NW
_RING = 8

_mesh = plsc.VectorSubcoreMesh(core_axis_name="c", subcore_axis_name="s")


@functools.partial(
    pl.kernel,
    mesh=_mesh,
    out_type=jax.ShapeDtypeStruct((_D, _B), jnp.float32),
    scratch_types=[
        pltpu.VMEM((_BPW,), jnp.int32),
        pltpu.VMEM((_RING, 32, 128), jnp.float32),
        pltpu.VMEM((4, 8, _BPW), jnp.float32),
        pltpu.SemaphoreType.DMA((_RING,)),
    ],
    compiler_params=pltpu.CompilerParams(
        use_tc_tiling_on_sc=True, needs_layout_passes=False
    ),
)
def _sc_gather(table_hbm, idx_hbm, out_hbm, idx_v, blk_v, cols_v, sems):
    wid = lax.axis_index("s") * _NC + lax.axis_index("c")
    base = wid * _BPW
    pltpu.sync_copy(idx_hbm.at[pl.ds(base, _BPW)], idx_v)

    lane = lax.broadcasted_iota(jnp.int32, (16,), 0)

    def read_idx(i):
        chunk = idx_v[pl.ds((i >> 4) * 16, 16)]
        return jnp.sum(jnp.where(lane == (i & 15), chunk, 0))

    def fetch(slot, v):
        c0 = (v >> 7) * 128
        for r in range(4):
            pltpu.async_copy(
                table_hbm.at[pl.ds(r * 8, 8), pl.ds(c0, 128)],
                blk_v.at[slot, pl.ds(r * 8, 8)],
                sems.at[slot],
            )

    def wait_block(slot):
        pltpu.make_async_copy(
            table_hbm.at[:, pl.ds(0, 128)],
            blk_v.at[slot],
            sems.at[slot],
        ).wait()

    for i in range(_RING):
        fetch(i, read_idx(jnp.int32(i)))

    def body(i, carry):
        slot = i & (_RING - 1)
        wait_block(slot)
        v = read_idx(i)
        l16 = jnp.full((16,), v & 127, dtype=jnp.int32)
        i16 = jnp.full((16,), i, dtype=jnp.int32)
        zero_dep = jnp.int32(0)
        for h in range(2):
            d16 = lane + (16 * h)
            vals = plsc.load_gather(blk_v.at[slot], [d16, l16])
            plsc.store_scatter(cols_v, [d16 >> 3, d16 & 7, i16], vals)
            zero_dep = zero_dep | (jnp.sum(plsc.bitcast(vals, jnp.int32)) & 0)
        nxt = i + _RING

        @pl.when(nxt < _BPW)
        def _():
            fetch(slot, read_idx(nxt) | zero_dep)

        return carry

    lax.fori_loop(0, _BPW, body, 0)

    for r in range(4):
        pltpu.sync_copy(
            cols_v.at[r], out_hbm.at[pl.ds(r * 8, 8), pl.ds(base, _BPW)]
        )


def kernel(x, embeddings):
    out_t = _sc_gather(embeddings.T, x.astype(jnp.int32))
    return out_t.T


# speedup vs baseline: 1.0243x; 1.0235x over previous
import functools

import jax
import jax.numpy as jnp
from jax import lax
from jax.experimental import pallas as pl
from jax.experimental.pallas import tpu as pltpu
from jax.experimental.pallas import tpu_sc as plsc

_VOCAB = 1000000
_D = 32
_B = 16384

_info = plsc.get_sparse_core_info()
_NC, _NS = _info.num_cores, _info.num_subcores
_NW = _NC * _NS
_BPW = _B // _NW
_RING = 8

_mesh = plsc.VectorSubcoreMesh(core_axis_name="c", subcore_axis_name="s")


@functools.partial(
    pl.kernel,
    mesh=_mesh,
    out_type=jax.ShapeDtypeStruct((_D, _B), jnp.float32),
    scratch_types=[
        pltpu.VMEM((_BPW,), jnp.int32),
        pltpu.VMEM((_RING, 4, 8, 128), jnp.float32),
        pltpu.VMEM((4, 8, _BPW), jnp.float32),
        pltpu.SemaphoreType.DMA((_RING,)),
    ],
    compiler_params=pltpu.CompilerParams(
        use_tc_tiling_on_sc=True, needs_layout_passes=False
    ),
)
def _sc_gather(table_hbm, idx_hbm, out_hbm, idx_v, blk_v, cols_v, sems):
    wid = lax.axis_index("s") * _NC + lax.axis_index("c")
    base = wid * _BPW
    pltpu.sync_copy(idx_hbm.at[pl.ds(base, _BPW)], idx_v)

    lane = lax.broadcasted_iota(jnp.int32, (16,), 0)

    def read_idx(i):
        chunk = idx_v[pl.ds((i >> 4) * 16, 16)]
        return jnp.sum(jnp.where(lane == (i & 15), chunk, 0))

    def fetch(slot, v):
        c0 = (v >> 7) * 128
        for r in range(4):
            pltpu.async_copy(
                table_hbm.at[pl.ds(r * 8, 8), pl.ds(c0, 128)],
                blk_v.at[slot, r],
                sems.at[slot],
            )

    def wait_block(slot):
        for r in range(4):
            pltpu.make_async_copy(
                table_hbm.at[pl.ds(0, 8), pl.ds(0, 128)],
                blk_v.at[slot, r],
                sems.at[slot],
            ).wait()

    for i in range(_RING):
        fetch(i, read_idx(jnp.int32(i)))

    def body(i, carry):
        slot = i & (_RING - 1)
        wait_block(slot)
        v = read_idx(i)
        l16 = jnp.full((16,), v & 127, dtype=jnp.int32)
        i16 = jnp.full((16,), i, dtype=jnp.int32)
        zero_dep = jnp.int32(0)
        for h in range(2):
            d16 = lane + (16 * h)
            r16 = d16 >> 3
            k16 = d16 & 7
            vals = plsc.load_gather(blk_v.at[slot], [r16, k16, l16])
            plsc.store_scatter(cols_v, [r16, k16, i16], vals)
            zero_dep = zero_dep | (jnp.sum(plsc.bitcast(vals, jnp.int32)) & 0)
        nxt = i + _RING

        @pl.when(nxt < _BPW)
        def _():
            fetch(slot, read_idx(nxt) | zero_dep)

        return carry

    lax.fori_loop(0, _BPW, body, 0)

    for r in range(4):
        pltpu.sync_copy(
            cols_v.at[r], out_hbm.at[pl.ds(r * 8, 8), pl.ds(base, _BPW)]
        )


def kernel(x, embeddings):
    out_t = _sc_gather(embeddings.T, x.astype(jnp.int32))
    return out_t.T
